# baseline (speedup 1.0000x reference)
import jax
import jax.numpy as jnp
from jax.experimental import pallas as pl
from jax.experimental.pallas import tpu as pltpu

_H = 200
_W = 200
_NF = 6
_NL = 16
_S = 40
_L = 1000
_LOGIT01 = -2.1972245773362196


def _occ_loss_kernel(traj_ref, gmask_ref, hbm_ref, out_ref, buf_ref, sem):
    for t in range(_NF):
        pltpu.make_async_copy(
            hbm_ref.at[:, t], buf_ref.at[t], sem.at[t]
        ).start()

    sf = jax.lax.broadcasted_iota(jnp.int32, (_S, _L), 0)
    lf = jax.lax.broadcasted_iota(jnp.int32, (_S, _L), 1)
    q = (
        (lf >= 200).astype(jnp.int32)
        + (lf >= 400).astype(jnp.int32)
        + (lf >= 600).astype(jnp.int32)
        + (lf >= 800).astype(jnp.int32)
    )
    rr = (5 * sf + q).astype(jnp.float32)
    cc = (lf - 200 * q).astype(jnp.float32)
    xg = jnp.trunc((cc - 100.0) * 0.5 + 0.25)
    yg = jnp.trunc((rr - 100.0) * 0.5 + 0.25)

    mask_sum = 0.0
    num = 0.0
    den = 0.0
    for t in range(_NF):
        pltpu.make_async_copy(
            hbm_ref.at[:, t], buf_ref.at[t], sem.at[t]
        ).wait()
        m = buf_ref[t]
        mx = jnp.max(m, axis=0)
        occ = (mx > _LOGIT01).astype(jnp.float32)
        mask_sum += jnp.sum(occ)

        futures = []
        if t > 0:
            futures.append(t - 1)
        if t == _NF - 1:
            futures.append(_NF - 1)
        for i in futures:
            px = traj_ref[i, 0]
            py = traj_ref[i, 1]
            dx = px - xg
            dy = py - yg
            d2 = dx * dx + dy * dy
            keep = (d2 < 25.0).astype(jnp.float32)
            w = occ * keep
            cnt = jnp.sum(w)
            gau = jnp.sum(jnp.exp(-0.5 * d2) * w)
            valid_g = (cnt > 0.0).astype(jnp.float32) * gmask_ref[i]
            num += 0.5 * gau / 2.507 * valid_g
            den += valid_g

    loss = jnp.where(den > 0.0, num / jnp.maximum(den, 1.0), 0.0)
    loss = jnp.where(mask_sum == 0.0, 0.0, loss)
    out_ref[0] = loss


def kernel(sdc_traj_all, sdc_planning_gt, sdc_planning_gt_mask, bev_mask, bev_target):
    traj = sdc_traj_all[0].astype(jnp.float32)
    gmask = (sdc_planning_gt_mask[0] != 0).astype(jnp.float32)
    bev = bev_mask.reshape(_NL, _NF, _S, _L)

    out = pl.pallas_call(
        _occ_loss_kernel,
        in_specs=[
            pl.BlockSpec(memory_space=pltpu.SMEM),
            pl.BlockSpec(memory_space=pltpu.SMEM),
            pl.BlockSpec(memory_space=pltpu.MemorySpace.HBM),
        ],
        out_specs=pl.BlockSpec(memory_space=pltpu.SMEM),
        out_shape=jax.ShapeDtypeStruct((1,), jnp.float32),
        scratch_shapes=[
            pltpu.VMEM((_NF, _NL, _S, _L), jnp.float32),
            pltpu.SemaphoreType.DMA((_NF,)),
        ],
    )(traj, gmask, bev)
    return out[0]
